# baseline (speedup 1.0000x reference)
import jax
import jax.numpy as jnp
from jax.experimental import pallas as pl
from jax.experimental.pallas import tpu as pltpu

_NUM_CLASSES = 1000


def _linear_kernel(x_ref, wt_ref, b_ref, o_ref, wbf_ref, ybuf_ref, sem_ref):
    i = pl.program_id(0)
    nsteps = pl.num_programs(0)
    tile_m = x_ref.shape[0]
    n = o_ref.shape[1]

    @pl.when(i == 0)
    def _():
        wbf_ref[...] = wt_ref[...].astype(jnp.bfloat16)

    x = x_ref[...].astype(jnp.bfloat16)
    acc = jnp.dot(x, wbf_ref[...], preferred_element_type=jnp.float32)
    ybuf_ref[pl.ds(i * tile_m, tile_m), :] = (acc + b_ref[...])[:, :n]

    @pl.when(i == nsteps - 1)
    def _():
        cp = pltpu.make_async_copy(ybuf_ref, o_ref, sem_ref)
        cp.start()
        cp.wait()


def kernel(x, wt_p, b_p):
    M, K = x.shape
    K_pad, N_pad = wt_p.shape
    n = min(_NUM_CLASSES, N_pad)

    tile_m = next(t for t in (512, 256, 128, 64, 8, 1) if M % t == 0)
    m_steps = M // tile_m

    cost = pl.CostEstimate(
        flops=2 * M * K_pad * N_pad,
        transcendentals=0,
        bytes_accessed=M * K * 4 + K_pad * N_pad * 4 + N_pad * 4 + M * n * 4,
    )

    return pl.pallas_call(
        _linear_kernel,
        out_shape=jax.ShapeDtypeStruct((M, n), x.dtype),
        grid=(m_steps,),
        in_specs=[
            pl.BlockSpec((tile_m, K), lambda i: (i, 0)),
            pl.BlockSpec((K_pad, N_pad), lambda i: (0, 0)),
            pl.BlockSpec((1, N_pad), lambda i: (0, 0)),
        ],
        out_specs=pl.BlockSpec(memory_space=pl.ANY),
        scratch_shapes=[
            pltpu.VMEM((K_pad, N_pad), jnp.bfloat16),
            pltpu.VMEM((M, n), jnp.float32),
            pltpu.SemaphoreType.DMA(()),
        ],
        compiler_params=pltpu.CompilerParams(
            dimension_semantics=("arbitrary",),
        ),
        cost_estimate=cost,
    )(x, wt_p, b_p)


# speedup vs baseline: 1.1099x; 1.1099x over previous
import jax
import jax.numpy as jnp
from jax.experimental import pallas as pl
from jax.experimental.pallas import tpu as pltpu

_NUM_CLASSES = 1000


def _linear_kernel(x_ref, wt_ref, b_ref, o_ref, wbf_ref):
    @pl.when(pl.program_id(0) == 0)
    def _():
        wbf_ref[...] = wt_ref[...].astype(jnp.bfloat16)

    n = o_ref.shape[1]
    x = x_ref[...].astype(jnp.bfloat16)
    acc = jnp.dot(x, wbf_ref[...], preferred_element_type=jnp.float32)
    o_ref[...] = (acc + b_ref[...])[:, :n]


def kernel(x, wt_p, b_p):
    M, K = x.shape
    K_pad, N_pad = wt_p.shape
    n = min(_NUM_CLASSES, N_pad)

    tile_m = next(t for t in (1024, 512, 256, 128, 64, 8, 1) if M % t == 0)
    m_steps = M // tile_m

    cost = pl.CostEstimate(
        flops=2 * M * K_pad * N_pad,
        transcendentals=0,
        bytes_accessed=M * K * 4 + K_pad * N_pad * 4 + N_pad * 4 + M * n * 4,
    )

    return pl.pallas_call(
        _linear_kernel,
        out_shape=jax.ShapeDtypeStruct((M, n), x.dtype),
        grid=(m_steps,),
        in_specs=[
            pl.BlockSpec((tile_m, K), lambda i: (i, 0)),
            pl.BlockSpec((K_pad, N_pad), lambda i: (0, 0)),
            pl.BlockSpec((1, N_pad), lambda i: (0, 0)),
        ],
        out_specs=pl.BlockSpec((tile_m, n), lambda i: (i, 0)),
        scratch_shapes=[pltpu.VMEM((K_pad, N_pad), jnp.bfloat16)],
        compiler_params=pltpu.CompilerParams(
            dimension_semantics=("arbitrary",),
        ),
        cost_estimate=cost,
    )(x, wt_p, b_p)
